# baseline (speedup 1.0000x reference)
import dataclasses
import functools

import jax
import jax.numpy as jnp
from jax import lax
from jax.experimental import pallas as pl
from jax.experimental.pallas import tpu as pltpu
from jax.experimental.pallas import tpu_sc as plsc

N_NODES = 10000
N_EDGES = 320000
D = 128

NC = 2
NS = 16
NW = NC * NS
LANES = 16
CH = 64
NB = 4
NCH = 160
S = 32
NST = NCH // S
EPT = NCH * CH
EPAD = NW * EPT
N_ACC = 10112
STRIPE = N_ACC // NS


def _scale_rows(buf, w_v, off):
    @pl.loop(0, CH)
    def _(r):
        iv = jnp.full((LANES,), 0, jnp.int32) + (off + r)
        wv = plsc.load_gather(w_v, [iv])
        for k in range(D // LANES):
            sl = pl.ds(k * LANES, LANES)
            buf[r, sl] = buf[r, sl] * wv


def _hist_update(hist, dst_v, off):
    ones = jnp.ones((LANES,), jnp.float32)
    for k in range(CH // LANES):
        idx = dst_v[pl.ds(off + k * LANES, LANES)]
        plsc.addupdate_scatter(hist, [idx], ones)


def _sc_body(x_hbm, srcs_hbm, dsts_hbm, ws_hbm, psum_hbm, cnt_hbm,
             src_v, dst_v, w_v, buf0, buf1, buf2, buf3, hist, acc_sh,
             sem_g0, sem_g1, sem_g2, sem_g3,
             sem_s0, sem_s1, sem_s2, sem_s3):
    bufs = (buf0, buf1, buf2, buf3)
    sems_g = (sem_g0, sem_g1, sem_g2, sem_g3)
    sems_s = (sem_s0, sem_s1, sem_s2, sem_s3)

    c = lax.axis_index("c")
    s = lax.axis_index("s")
    wid = c * NS + s

    zeros16 = jnp.zeros((LANES,), jnp.float32)

    base = s * STRIPE
    with jax.named_scope("sc_init"):
        @pl.loop(0, N_ACC, step=LANES)
        def _(i):
            hist[pl.ds(i, LANES)] = zeros16

        @pl.loop(0, CH)
        def _(r):
            for k in range(D // LANES):
                buf0[r, pl.ds(k * LANES, LANES)] = zeros16
        for k in range(STRIPE // CH):
            pltpu.sync_copy(buf0, acc_sh.at[pl.ds(base + k * CH, CH)])
        rem = STRIPE % CH
        pltpu.sync_copy(buf0.at[pl.ds(0, rem)],
                        acc_sh.at[pl.ds(base + STRIPE - rem, rem)])

    with jax.named_scope("sc_barrier0"):
        plsc.subcore_barrier()

    def phase(j, buf, sem_g, sem_s):
        off = j * CH
        gidx = src_v.at[pl.ds(off, CH)]
        sidx = dst_v.at[pl.ds(off, CH)]
        pltpu.make_async_copy(x_hbm.at[gidx], buf, sem_g).wait()
        _scale_rows(buf, w_v, off)
        pltpu.async_copy(buf, acc_sh.at[sidx], sem_s, add=True)
        _hist_update(hist, dst_v, off)
        pltpu.make_async_copy(buf, acc_sh.at[sidx], sem_s).wait()

        @pl.when(j + NB < S)
        def _():
            pltpu.async_copy(
                x_hbm.at[src_v.at[pl.ds((j + NB) * CH, CH)]], buf, sem_g)

    @pl.loop(0, NST)
    def _(st):
        with jax.named_scope("sc_stage"):
            pltpu.sync_copy(srcs_hbm.at[wid, st], src_v)
            pltpu.sync_copy(dsts_hbm.at[wid, st], dst_v)
            pltpu.sync_copy(ws_hbm.at[wid, st], w_v)

            for b in range(NB):
                pltpu.async_copy(
                    x_hbm.at[src_v.at[pl.ds(b * CH, CH)]], bufs[b],
                    sems_g[b])

        with jax.named_scope("sc_mainloop"):
            @pl.loop(0, S, step=NB)
            def _(j):
                for b in range(NB):
                    phase(j + b, bufs[b], sems_g[b], sems_s[b])

    with jax.named_scope("sc_barrier1"):
        plsc.subcore_barrier()

    with jax.named_scope("sc_readout"):
        pltpu.sync_copy(acc_sh.at[pl.ds(base, STRIPE)],
                        psum_hbm.at[c, pl.ds(base, STRIPE)])
        pltpu.sync_copy(hist, cnt_hbm.at[wid])


_sc_cp = pltpu.CompilerParams()
if "needs_layout_passes" in pltpu.CompilerParams.__dataclass_fields__:
    _sc_cp = dataclasses.replace(_sc_cp, needs_layout_passes=False)

_sc_aggregate = functools.partial(
    pl.kernel,
    compiler_params=_sc_cp,
    out_type=[
        jax.ShapeDtypeStruct((NC, N_ACC, D), jnp.float32),
        jax.ShapeDtypeStruct((NW, N_ACC), jnp.float32),
    ],
    mesh=plsc.VectorSubcoreMesh(core_axis_name="c", subcore_axis_name="s"),
    scratch_types=[
        pltpu.VMEM((S * CH,), jnp.int32),
        pltpu.VMEM((S * CH,), jnp.int32),
        pltpu.VMEM((S * CH,), jnp.float32),
        pltpu.VMEM((CH, D), jnp.float32),
        pltpu.VMEM((CH, D), jnp.float32),
        pltpu.VMEM((CH, D), jnp.float32),
        pltpu.VMEM((CH, D), jnp.float32),
        pltpu.VMEM((N_ACC,), jnp.float32),
        pltpu.VMEM_SHARED((N_ACC, D), jnp.float32),
        pltpu.SemaphoreType.DMA,
        pltpu.SemaphoreType.DMA,
        pltpu.SemaphoreType.DMA,
        pltpu.SemaphoreType.DMA,
        pltpu.SemaphoreType.DMA,
        pltpu.SemaphoreType.DMA,
        pltpu.SemaphoreType.DMA,
        pltpu.SemaphoreType.DMA,
    ],
)(_sc_body)


BLK = 2000


def _tc_body(p_ref, c_ref, w_ref, b_ref, o_ref):
    ssum = p_ref[0] + p_ref[1]
    cnt = jnp.sum(c_ref[...], axis=1, keepdims=True)
    h = jnp.tanh(ssum / jnp.maximum(cnt, 1.0))
    o_ref[...] = lax.dot_general(
        h, w_ref[...], (((1,), (1,)), ((), ())),
        preferred_element_type=jnp.float32) + b_ref[...]


_tc_finish = pl.pallas_call(
    _tc_body,
    grid=(N_NODES // BLK,),
    in_specs=[
        pl.BlockSpec((NC, BLK, D), lambda i: (0, i, 0)),
        pl.BlockSpec((BLK, NW), lambda i: (i, 0)),
        pl.BlockSpec((D, D), lambda i: (0, 0)),
        pl.BlockSpec((1, D), lambda i: (0, 0)),
    ],
    out_specs=pl.BlockSpec((BLK, D), lambda i: (i, 0)),
    out_shape=jax.ShapeDtypeStruct((N_NODES, D), jnp.float32),
)


def kernel(x, edge_index, edge_weight, W, b):
    src = edge_index[0].astype(jnp.int32)
    dst = edge_index[1].astype(jnp.int32)
    w = edge_weight.astype(jnp.float32)
    pad = EPAD - N_EDGES
    src_p = jnp.concatenate(
        [src, jnp.zeros((pad,), jnp.int32)]).reshape(NW, NST, S * CH)
    dst_p = jnp.concatenate(
        [dst, jnp.full((pad,), N_NODES, jnp.int32)]).reshape(NW, NST, S * CH)
    w_p = jnp.concatenate(
        [w, jnp.zeros((pad,), jnp.float32)]).reshape(NW, NST, S * CH)
    psum, cnt = _sc_aggregate(x, src_p, dst_p, w_p)
    return _tc_finish(psum, cnt.T, W, b.reshape(1, D))


# speedup vs baseline: 1.8896x; 1.8896x over previous
import dataclasses
import functools

import jax
import jax.numpy as jnp
from jax import lax
from jax.experimental import pallas as pl
from jax.experimental.pallas import tpu as pltpu
from jax.experimental.pallas import tpu_sc as plsc

N_NODES = 10000
N_EDGES = 320000
D = 128

NC = 2
NS = 16
NW = NC * NS
LANES = 16
CH = 64
NB = 4
NCH = 160
S = 32
NST = NCH // S
EPT = NCH * CH
EPAD = NW * EPT
N_ACC = 10112
STRIPE = N_ACC // NS


def _scale_rows(buf, w_v, off):
    @pl.loop(0, CH)
    def _(r):
        iv = jnp.full((LANES,), 0, jnp.int32) + (off + r)
        wv = plsc.load_gather(w_v, [iv])
        for k in range(D // LANES):
            sl = pl.ds(k * LANES, LANES)
            buf[r, sl] = buf[r, sl] * wv


def _hist_update(hist, dst_v, off):
    ones = jnp.ones((LANES,), jnp.float32)
    for k in range(CH // LANES):
        idx = dst_v[pl.ds(off + k * LANES, LANES)]
        plsc.addupdate_scatter(hist, [idx], ones)


def _sc_body(x_hbm, srcs_hbm, dsts_hbm, ws_hbm, psum_hbm, cnt_hbm,
             src_v, dst_v, w_v, buf0, buf1, buf2, buf3, hist, acc_sh,
             sem_g0, sem_g1, sem_g2, sem_g3,
             sem_s0, sem_s1, sem_s2, sem_s3):
    bufs = (buf0, buf1, buf2, buf3)
    sems_g = (sem_g0, sem_g1, sem_g2, sem_g3)
    sems_s = (sem_s0, sem_s1, sem_s2, sem_s3)

    c = lax.axis_index("c")
    s = lax.axis_index("s")
    wid = c * NS + s

    zeros16 = jnp.zeros((LANES,), jnp.float32)

    base = s * STRIPE
    with jax.named_scope("sc_init"):
        @pl.loop(0, N_ACC, step=LANES)
        def _(i):
            hist[pl.ds(i, LANES)] = zeros16

        @pl.loop(0, CH)
        def _(r):
            for k in range(D // LANES):
                buf0[r, pl.ds(k * LANES, LANES)] = zeros16
        for k in range(STRIPE // CH):
            pltpu.sync_copy(buf0, acc_sh.at[pl.ds(base + k * CH, CH)])
        rem = STRIPE % CH
        pltpu.sync_copy(buf0.at[pl.ds(0, rem)],
                        acc_sh.at[pl.ds(base + STRIPE - rem, rem)])

    with jax.named_scope("sc_barrier0"):
        plsc.subcore_barrier()

    def phase(j, buf, sem_g, sem_s):
        off = j * CH
        gidx = src_v.at[pl.ds(off, CH)]
        sidx = dst_v.at[pl.ds(off, CH)]
        pltpu.make_async_copy(x_hbm.at[gidx], buf, sem_g).wait()
        _scale_rows(buf, w_v, off)
        pltpu.async_copy(buf, acc_sh.at[sidx], sem_s, add=True)
        _hist_update(hist, dst_v, off)
        pltpu.make_async_copy(buf, acc_sh.at[sidx], sem_s).wait()

        @pl.when(j + NB < S)
        def _():
            pltpu.async_copy(
                x_hbm.at[src_v.at[pl.ds((j + NB) * CH, CH)]], buf, sem_g)

    @pl.loop(0, NST)
    def _(st):
        with jax.named_scope("sc_stage"):
            pltpu.sync_copy(srcs_hbm.at[wid, st], src_v)
            pltpu.sync_copy(dsts_hbm.at[wid, st], dst_v)
            pltpu.sync_copy(ws_hbm.at[wid, st], w_v)

            for b in range(NB):
                pltpu.async_copy(
                    x_hbm.at[src_v.at[pl.ds(b * CH, CH)]], bufs[b],
                    sems_g[b])

        with jax.named_scope("sc_mainloop"):
            @pl.loop(0, S, step=NB)
            def _(j):
                for b in range(NB):
                    phase(j + b, bufs[b], sems_g[b], sems_s[b])

    with jax.named_scope("sc_barrier1"):
        plsc.subcore_barrier()

    with jax.named_scope("sc_readout"):
        pltpu.sync_copy(acc_sh.at[pl.ds(base, STRIPE)],
                        psum_hbm.at[c, pl.ds(base, STRIPE)])
        pltpu.sync_copy(hist, cnt_hbm.at[wid])


_sc_cp = pltpu.CompilerParams()
if "needs_layout_passes" in pltpu.CompilerParams.__dataclass_fields__:
    _sc_cp = dataclasses.replace(_sc_cp, needs_layout_passes=False)

_sc_aggregate = functools.partial(
    pl.kernel,
    compiler_params=_sc_cp,
    out_type=[
        jax.ShapeDtypeStruct((NC, N_ACC, D), jnp.float32),
        jax.ShapeDtypeStruct((NW, N_ACC), jnp.float32),
    ],
    mesh=plsc.VectorSubcoreMesh(core_axis_name="c", subcore_axis_name="s"),
    scratch_types=[
        pltpu.VMEM((S * CH,), jnp.int32),
        pltpu.VMEM((S * CH,), jnp.int32),
        pltpu.VMEM((S * CH,), jnp.float32),
        pltpu.VMEM((CH, D), jnp.float32),
        pltpu.VMEM((CH, D), jnp.float32),
        pltpu.VMEM((CH, D), jnp.float32),
        pltpu.VMEM((CH, D), jnp.float32),
        pltpu.VMEM((N_ACC,), jnp.float32),
        pltpu.VMEM_SHARED((N_ACC, D), jnp.float32),
        pltpu.SemaphoreType.DMA,
        pltpu.SemaphoreType.DMA,
        pltpu.SemaphoreType.DMA,
        pltpu.SemaphoreType.DMA,
        pltpu.SemaphoreType.DMA,
        pltpu.SemaphoreType.DMA,
        pltpu.SemaphoreType.DMA,
        pltpu.SemaphoreType.DMA,
    ],
)(_sc_body)


BLK = 2000


def _tc_body(p_ref, c_ref, w_ref, b_ref, o_ref):
    ssum = p_ref[0] + p_ref[1]
    cnt = jnp.sum(c_ref[...], axis=1, keepdims=True)
    h = jnp.tanh(ssum / jnp.maximum(cnt, 1.0))
    o_ref[...] = lax.dot_general(
        h, w_ref[...], (((1,), (1,)), ((), ())),
        preferred_element_type=jnp.float32) + b_ref[...]


_tc_finish = pl.pallas_call(
    _tc_body,
    grid=(N_NODES // BLK,),
    in_specs=[
        pl.BlockSpec((NC, BLK, D), lambda i: (0, i, 0)),
        pl.BlockSpec((BLK, NW), lambda i: (i, 0)),
        pl.BlockSpec((D, D), lambda i: (0, 0)),
        pl.BlockSpec((1, D), lambda i: (0, 0)),
    ],
    out_specs=pl.BlockSpec((BLK, D), lambda i: (i, 0)),
    out_shape=jax.ShapeDtypeStruct((N_NODES, D), jnp.float32),
)


def kernel(x, edge_index, edge_weight, W, b):
    src = edge_index[0].astype(jnp.int32)
    dst = edge_index[1].astype(jnp.int32)
    w = edge_weight.astype(jnp.float32)
    pad = EPAD - N_EDGES
    pad_i = jnp.arange(pad, dtype=jnp.int32)
    src_p = jnp.concatenate(
        [src, pad_i % N_NODES]).reshape(NW, NST, S * CH)
    dst_p = jnp.concatenate(
        [dst, N_NODES + pad_i % (N_ACC - N_NODES)]).reshape(NW, NST, S * CH)
    w_p = jnp.concatenate(
        [w, jnp.zeros((pad,), jnp.float32)]).reshape(NW, NST, S * CH)
    psum, cnt = _sc_aggregate(x, src_p, dst_p, w_p)
    return _tc_finish(psum, cnt.T, W, b.reshape(1, D))


# speedup vs baseline: 1.9049x; 1.0081x over previous
import dataclasses
import functools

import jax
import jax.numpy as jnp
from jax import lax
from jax.experimental import pallas as pl
from jax.experimental.pallas import tpu as pltpu
from jax.experimental.pallas import tpu_sc as plsc

N_NODES = 10000
N_EDGES = 320000
D = 128

NC = 2
NS = 16
NW = NC * NS
LANES = 16
CH = 64
NB = 4
NCH = 160
S = 32
NST = NCH // S
EPT = NCH * CH
EPAD = NW * EPT
N_ACC = 10112
STRIPE = N_ACC // NS


def _scale_rows(buf, w_v, off):
    @pl.loop(0, CH)
    def _(r):
        iv = jnp.full((LANES,), 0, jnp.int32) + (off + r)
        wv = plsc.load_gather(w_v, [iv])
        for k in range(D // LANES):
            sl = pl.ds(k * LANES, LANES)
            buf[r, sl] = buf[r, sl] * wv


def _hist_update(hist, dst_v, off):
    ones = jnp.ones((LANES,), jnp.float32)
    for k in range(CH // LANES):
        idx = dst_v[pl.ds(off + k * LANES, LANES)]
        plsc.addupdate_scatter(hist, [idx], ones)


WSZ = S * CH
G_FULL = N_EDGES // WSZ
G_TAIL = NW * NST - G_FULL


def _sc_body(x_hbm, srcs_hbm, dsts_hbm, ws_hbm,
             srct_hbm, dstt_hbm, wt_hbm, psum_hbm, cnt_hbm,
             src_v, dst_v, w_v, buf0, buf1, buf2, buf3, hist, acc_sh,
             sem_g0, sem_g1, sem_g2, sem_g3,
             sem_s0, sem_s1, sem_s2, sem_s3):
    bufs = (buf0, buf1, buf2, buf3)
    sems_g = (sem_g0, sem_g1, sem_g2, sem_g3)
    sems_s = (sem_s0, sem_s1, sem_s2, sem_s3)

    c = lax.axis_index("c")
    s = lax.axis_index("s")
    wid = c * NS + s

    zeros16 = jnp.zeros((LANES,), jnp.float32)

    base = s * STRIPE
    with jax.named_scope("sc_init"):
        @pl.loop(0, N_ACC, step=LANES)
        def _(i):
            hist[pl.ds(i, LANES)] = zeros16

        @pl.loop(0, CH)
        def _(r):
            for k in range(D // LANES):
                buf0[r, pl.ds(k * LANES, LANES)] = zeros16
        for k in range(STRIPE // CH):
            pltpu.sync_copy(buf0, acc_sh.at[pl.ds(base + k * CH, CH)])
        rem = STRIPE % CH
        pltpu.sync_copy(buf0.at[pl.ds(0, rem)],
                        acc_sh.at[pl.ds(base + STRIPE - rem, rem)])

    with jax.named_scope("sc_barrier0"):
        plsc.subcore_barrier()

    def phase(j, buf, sem_g, sem_s):
        off = j * CH
        gidx = src_v.at[pl.ds(off, CH)]
        sidx = dst_v.at[pl.ds(off, CH)]
        pltpu.make_async_copy(x_hbm.at[gidx], buf, sem_g).wait()
        _scale_rows(buf, w_v, off)
        pltpu.async_copy(buf, acc_sh.at[sidx], sem_s, add=True)
        _hist_update(hist, dst_v, off)
        pltpu.make_async_copy(buf, acc_sh.at[sidx], sem_s).wait()

        @pl.when(j + NB < S)
        def _():
            pltpu.async_copy(
                x_hbm.at[src_v.at[pl.ds((j + NB) * CH, CH)]], buf, sem_g)

    @pl.loop(0, NST)
    def _(st):
        with jax.named_scope("sc_stage"):
            g = wid * NST + st

            @pl.when(g < G_FULL)
            def _():
                off = g * WSZ
                pltpu.sync_copy(srcs_hbm.at[pl.ds(off, WSZ)], src_v)
                pltpu.sync_copy(dsts_hbm.at[pl.ds(off, WSZ)], dst_v)
                pltpu.sync_copy(ws_hbm.at[pl.ds(off, WSZ)], w_v)

            @pl.when(g >= G_FULL)
            def _():
                off = (g - G_FULL) * WSZ
                pltpu.sync_copy(srct_hbm.at[pl.ds(off, WSZ)], src_v)
                pltpu.sync_copy(dstt_hbm.at[pl.ds(off, WSZ)], dst_v)
                pltpu.sync_copy(wt_hbm.at[pl.ds(off, WSZ)], w_v)

            for b in range(NB):
                pltpu.async_copy(
                    x_hbm.at[src_v.at[pl.ds(b * CH, CH)]], bufs[b],
                    sems_g[b])

        with jax.named_scope("sc_mainloop"):
            @pl.loop(0, S, step=NB)
            def _(j):
                for b in range(NB):
                    phase(j + b, bufs[b], sems_g[b], sems_s[b])

    with jax.named_scope("sc_barrier1"):
        plsc.subcore_barrier()

    with jax.named_scope("sc_readout"):
        pltpu.sync_copy(acc_sh.at[pl.ds(base, STRIPE)],
                        psum_hbm.at[c, pl.ds(base, STRIPE)])
        pltpu.sync_copy(hist, cnt_hbm.at[wid])


_sc_cp = pltpu.CompilerParams()
if "needs_layout_passes" in pltpu.CompilerParams.__dataclass_fields__:
    _sc_cp = dataclasses.replace(_sc_cp, needs_layout_passes=False)

_sc_aggregate = functools.partial(
    pl.kernel,
    compiler_params=_sc_cp,
    out_type=[
        jax.ShapeDtypeStruct((NC, N_ACC, D), jnp.float32),
        jax.ShapeDtypeStruct((NW, N_ACC), jnp.float32),
    ],
    mesh=plsc.VectorSubcoreMesh(core_axis_name="c", subcore_axis_name="s"),
    scratch_types=[
        pltpu.VMEM((S * CH,), jnp.int32),
        pltpu.VMEM((S * CH,), jnp.int32),
        pltpu.VMEM((S * CH,), jnp.float32),
        pltpu.VMEM((CH, D), jnp.float32),
        pltpu.VMEM((CH, D), jnp.float32),
        pltpu.VMEM((CH, D), jnp.float32),
        pltpu.VMEM((CH, D), jnp.float32),
        pltpu.VMEM((N_ACC,), jnp.float32),
        pltpu.VMEM_SHARED((N_ACC, D), jnp.float32),
        pltpu.SemaphoreType.DMA,
        pltpu.SemaphoreType.DMA,
        pltpu.SemaphoreType.DMA,
        pltpu.SemaphoreType.DMA,
        pltpu.SemaphoreType.DMA,
        pltpu.SemaphoreType.DMA,
        pltpu.SemaphoreType.DMA,
        pltpu.SemaphoreType.DMA,
    ],
)(_sc_body)


BLK = 2000


def _tc_body(p_ref, c_ref, w_ref, b_ref, o_ref):
    ssum = p_ref[0] + p_ref[1]
    h = jnp.tanh(ssum / jnp.maximum(c_ref[...], 1.0))
    o_ref[...] = lax.dot_general(
        h, w_ref[...], (((1,), (1,)), ((), ())),
        preferred_element_type=jnp.float32) + b_ref[...]


_tc_finish = pl.pallas_call(
    _tc_body,
    grid=(N_NODES // BLK,),
    in_specs=[
        pl.BlockSpec((NC, BLK, D), lambda i: (0, i, 0)),
        pl.BlockSpec((BLK, 1), lambda i: (i, 0)),
        pl.BlockSpec((D, D), lambda i: (0, 0)),
        pl.BlockSpec((1, D), lambda i: (0, 0)),
    ],
    out_specs=pl.BlockSpec((BLK, D), lambda i: (i, 0)),
    out_shape=jax.ShapeDtypeStruct((N_NODES, D), jnp.float32),
)


def kernel(x, edge_index, edge_weight, W, b):
    src = edge_index[0].astype(jnp.int32)
    dst = edge_index[1].astype(jnp.int32)
    w = edge_weight.astype(jnp.float32)
    tail0 = G_FULL * WSZ
    pad = EPAD - N_EDGES
    pad_i = jnp.arange(pad, dtype=jnp.int32)
    src_t = jnp.concatenate([src[tail0:], pad_i % N_NODES])
    dst_t = jnp.concatenate(
        [dst[tail0:], N_NODES + pad_i % (N_ACC - N_NODES)])
    w_t = jnp.concatenate([w[tail0:], jnp.zeros((pad,), jnp.float32)])
    psum, cnt = _sc_aggregate(x, src, dst, w, src_t, dst_t, w_t)
    cnt_col = jnp.sum(cnt, axis=0)[:, None]
    return _tc_finish(psum, cnt_col, W, b.reshape(1, D))
